# Initial kernel scaffold; baseline (speedup 1.0000x reference)
import jax
import jax.numpy as jnp
from jax.experimental import pallas as pl


def kernel(x, edge_index, edge_attr, edge_weight, batch, params):
    raise NotImplementedError("write your pallas kernel here")


# speedup vs baseline: 1.0068x; 1.0068x over previous
import functools

import jax
import jax.numpy as jnp
from jax.experimental import pallas as pl

_N = 10000
_H = 512
_BN_EPS = 1e-5
_RT = 1000
_G = _N // _RT


def _row_spec(k):
    return pl.BlockSpec((_RT, k), lambda i: (i, 0))


def _full_spec(r, c):
    return pl.BlockSpec((r, c), lambda i: (0, 0))


def _gin_in_kernel(h_ref, a_ref, w_ref, b_ref, o_ref):
    y = h_ref[...] + a_ref[...]
    o_ref[...] = (
        jnp.dot(y, w_ref[...], preferred_element_type=jnp.float32) + b_ref[...]
    )


def _stats_kernel(z_ref, o_ref, *, relu):
    z = z_ref[...]
    if relu:
        z = jnp.maximum(z, 0.0)
    s = jnp.sum(z, axis=0, keepdims=True)
    ss = jnp.sum(z * z, axis=0, keepdims=True)
    val = jnp.concatenate([s, ss, jnp.zeros((6, z.shape[1]), jnp.float32)], axis=0)

    @pl.when(pl.program_id(0) == 0)
    def _():
        o_ref[...] = jnp.zeros_like(o_ref)

    o_ref[...] += val


def _bn_norm(z, st_ref, g, be):
    mu = st_ref[0:1, :] / _N
    var = st_ref[1:2, :] / _N - mu * mu
    inv = jax.lax.rsqrt(var + _BN_EPS)
    return g * (z - mu) * inv + be


def _gin_out_kernel(z_ref, st_ref, g_ref, be_ref, w2_ref, b2_ref, o_ref, *, relu_out):
    zn = _bn_norm(z_ref[...], st_ref, g_ref[...], be_ref[...])
    r = jnp.maximum(zn, 0.0)
    o = jnp.dot(r, w2_ref[...], preferred_element_type=jnp.float32) + b2_ref[...]
    if relu_out:
        o = jnp.maximum(o, 0.0)
    o_ref[...] = o


def _gat0_kernel(h_ref, st_ref, g_ref, be_ref, wc_ref, o_ref):
    r = jnp.maximum(h_ref[...], 0.0)
    z = _bn_norm(r, st_ref, g_ref[...], be_ref[...])
    o_ref[...] = jnp.dot(z, wc_ref[...], preferred_element_type=jnp.float32)


def _gat_kernel(h_ref, bp_ref, wc_ref, o_ref):
    z = jnp.maximum(h_ref[...] + bp_ref[...], 0.0)
    o_ref[...] = jnp.dot(z, wc_ref[...], preferred_element_type=jnp.float32)


def _bias_relu_kernel(h_ref, b_ref, o_ref):
    o_ref[...] = jnp.maximum(h_ref[...] + b_ref[...], 0.0)


def _call(kfn, out_cols, in_arrays, in_specs):
    return pl.pallas_call(
        kfn,
        grid=(_G,),
        in_specs=in_specs,
        out_specs=_row_spec(out_cols),
        out_shape=jax.ShapeDtypeStruct((_N, out_cols), jnp.float32),
    )(*in_arrays)


def _stats(z, relu):
    return pl.pallas_call(
        functools.partial(_stats_kernel, relu=relu),
        grid=(_G,),
        in_specs=[_row_spec(_H)],
        out_specs=_full_spec(8, _H),
        out_shape=jax.ShapeDtypeStruct((8, _H), jnp.float32),
    )(z)


def kernel(x, edge_index, edge_attr, edge_weight, batch, params):
    src = edge_index[0]
    dst = edge_index[1]
    h = x
    for l in range(3):
        p = params["gin"][l]
        k = h.shape[1]
        agg = jax.ops.segment_sum(h[src], dst, num_segments=_N)
        z = _call(
            _gin_in_kernel,
            _H,
            (h, agg, p["W1"], p["b1"].reshape(1, _H)),
            [_row_spec(k), _row_spec(k), _full_spec(k, _H), _full_spec(1, _H)],
        )
        st = _stats(z, relu=False)
        h = _call(
            functools.partial(_gin_out_kernel, relu_out=l < 2),
            _H,
            (z, st, p["g1"].reshape(1, _H), p["be1"].reshape(1, _H), p["W2"],
             p["b2"].reshape(1, _H)),
            [_row_spec(_H), _full_spec(8, _H), _full_spec(1, _H), _full_spec(1, _H),
             _full_spec(_H, _H), _full_spec(1, _H)],
        )

    loop = jnp.arange(_N, dtype=src.dtype)
    src2 = jnp.concatenate([src, loop])
    dst2 = jnp.concatenate([dst, loop])
    st = _stats(h, relu=True)
    bprev = None
    for l in range(3):
        p = params["gat"][l]
        wc = jnp.concatenate(
            [p["W"], p["W"] @ p["as"][:, None], p["W"] @ p["ad"][:, None],
             jnp.zeros((_H, 126), jnp.float32)], axis=1)
        if l == 0:
            o = _call(
                _gat0_kernel,
                _H + 128,
                (h, st, params["bn1_g"].reshape(1, _H),
                 params["bn1_b"].reshape(1, _H), wc),
                [_row_spec(_H), _full_spec(8, _H), _full_spec(1, _H),
                 _full_spec(1, _H), _full_spec(_H, _H + 128)],
            )
        else:
            o = _call(
                _gat_kernel,
                _H + 128,
                (h, bprev.reshape(1, _H), wc),
                [_row_spec(_H), _full_spec(1, _H), _full_spec(_H, _H + 128)],
            )
        hw = o[:, :_H]
        a_src = o[:, _H]
        a_dst = o[:, _H + 1]
        e = jax.nn.leaky_relu(a_src[src2] + a_dst[dst2], 0.2)
        m = jax.ops.segment_max(e, dst2, num_segments=_N)
        ex = jnp.exp(e - m[dst2])
        s = jax.ops.segment_sum(ex, dst2, num_segments=_N)
        alpha = ex / (s[dst2] + 1e-16)
        h = jax.ops.segment_sum(hw[src2] * alpha[:, None], dst2, num_segments=_N)
        bprev = p["b"]
    return _call(
        _bias_relu_kernel,
        _H,
        (h, bprev.reshape(1, _H)),
        [_row_spec(_H), _full_spec(1, _H)],
    )


# speedup vs baseline: 1.0505x; 1.0434x over previous
import functools

import jax
import jax.numpy as jnp
from jax.experimental import pallas as pl

_N = 10000
_H = 512
_BN_EPS = 1e-5
_RT = 1000
_G = _N // _RT


def _row_spec(k):
    return pl.BlockSpec((_RT, k), lambda i: (i, 0))


def _full_spec(r, c):
    return pl.BlockSpec((r, c), lambda i: (0, 0))


def _gin_in_kernel(h_ref, a_ref, w_ref, b_ref, o_ref):
    y = h_ref[...] + a_ref[...]
    o_ref[...] = (
        jnp.dot(y, w_ref[...], preferred_element_type=jnp.float32) + b_ref[...]
    )


def _stats_kernel(z_ref, o_ref, *, relu):
    z = z_ref[...]
    if relu:
        z = jnp.maximum(z, 0.0)
    s = jnp.sum(z, axis=0, keepdims=True)
    ss = jnp.sum(z * z, axis=0, keepdims=True)
    val = jnp.concatenate([s, ss, jnp.zeros((6, z.shape[1]), jnp.float32)], axis=0)

    @pl.when(pl.program_id(0) == 0)
    def _():
        o_ref[...] = jnp.zeros_like(o_ref)

    o_ref[...] += val


def _bn_norm(z, st_ref, g, be):
    mu = st_ref[0:1, :] / _N
    var = st_ref[1:2, :] / _N - mu * mu
    inv = jax.lax.rsqrt(var + _BN_EPS)
    return g * (z - mu) * inv + be


def _gin_out_kernel(z_ref, st_ref, g_ref, be_ref, w2_ref, b2_ref, o_ref, *, relu_out):
    zn = _bn_norm(z_ref[...], st_ref, g_ref[...], be_ref[...])
    r = jnp.maximum(zn, 0.0)
    o = jnp.dot(r, w2_ref[...], preferred_element_type=jnp.float32) + b2_ref[...]
    if relu_out:
        o = jnp.maximum(o, 0.0)
    o_ref[...] = o


def _gat0_kernel(h_ref, st_ref, g_ref, be_ref, wc_ref, o_ref):
    r = jnp.maximum(h_ref[...], 0.0)
    z = _bn_norm(r, st_ref, g_ref[...], be_ref[...])
    o_ref[...] = jnp.dot(z, wc_ref[...], preferred_element_type=jnp.float32)


def _gat_kernel(h_ref, bp_ref, wc_ref, o_ref):
    z = jnp.maximum(h_ref[...] + bp_ref[...], 0.0)
    o_ref[...] = jnp.dot(z, wc_ref[...], preferred_element_type=jnp.float32)


def _bias_relu_kernel(h_ref, b_ref, o_ref):
    o_ref[...] = jnp.maximum(h_ref[...] + b_ref[...], 0.0)


def _call(kfn, out_cols, in_arrays, in_specs):
    return pl.pallas_call(
        kfn,
        grid=(_G,),
        in_specs=in_specs,
        out_specs=_row_spec(out_cols),
        out_shape=jax.ShapeDtypeStruct((_N, out_cols), jnp.float32),
    )(*in_arrays)


def _stats(z, relu):
    return pl.pallas_call(
        functools.partial(_stats_kernel, relu=relu),
        grid=(_G,),
        in_specs=[_row_spec(_H)],
        out_specs=_full_spec(8, _H),
        out_shape=jax.ShapeDtypeStruct((8, _H), jnp.float32),
    )(z)


def kernel(x, edge_index, edge_attr, edge_weight, batch, params):
    order = jnp.argsort(edge_index[1])
    src = edge_index[0][order]
    dst = edge_index[1][order]
    h = x
    for l in range(3):
        p = params["gin"][l]
        k = h.shape[1]
        agg = jax.ops.segment_sum(h[src], dst, num_segments=_N,
                                  indices_are_sorted=True)
        z = _call(
            _gin_in_kernel,
            _H,
            (h, agg, p["W1"], p["b1"].reshape(1, _H)),
            [_row_spec(k), _row_spec(k), _full_spec(k, _H), _full_spec(1, _H)],
        )
        st = _stats(z, relu=False)
        h = _call(
            functools.partial(_gin_out_kernel, relu_out=l < 2),
            _H,
            (z, st, p["g1"].reshape(1, _H), p["be1"].reshape(1, _H), p["W2"],
             p["b2"].reshape(1, _H)),
            [_row_spec(_H), _full_spec(8, _H), _full_spec(1, _H), _full_spec(1, _H),
             _full_spec(_H, _H), _full_spec(1, _H)],
        )

    loop = jnp.arange(_N, dtype=src.dtype)
    order2 = jnp.argsort(jnp.concatenate([dst, loop]))
    src2 = jnp.concatenate([src, loop])[order2]
    dst2 = jnp.concatenate([dst, loop])[order2]
    st = _stats(h, relu=True)
    bprev = None
    for l in range(3):
        p = params["gat"][l]
        wc = jnp.concatenate(
            [p["W"], p["W"] @ p["as"][:, None], p["W"] @ p["ad"][:, None],
             jnp.zeros((_H, 126), jnp.float32)], axis=1)
        if l == 0:
            o = _call(
                _gat0_kernel,
                _H + 128,
                (h, st, params["bn1_g"].reshape(1, _H),
                 params["bn1_b"].reshape(1, _H), wc),
                [_row_spec(_H), _full_spec(8, _H), _full_spec(1, _H),
                 _full_spec(1, _H), _full_spec(_H, _H + 128)],
            )
        else:
            o = _call(
                _gat_kernel,
                _H + 128,
                (h, bprev.reshape(1, _H), wc),
                [_row_spec(_H), _full_spec(1, _H), _full_spec(_H, _H + 128)],
            )
        hw = o[:, :_H]
        a_src = o[:, _H]
        a_dst = o[:, _H + 1]
        e = jax.nn.leaky_relu(a_src[src2] + a_dst[dst2], 0.2)
        m = jax.ops.segment_max(e, dst2, num_segments=_N, indices_are_sorted=True)
        ex = jnp.exp(e - m[dst2])
        s = jax.ops.segment_sum(ex, dst2, num_segments=_N, indices_are_sorted=True)
        alpha = ex / (s[dst2] + 1e-16)
        h = jax.ops.segment_sum(hw[src2] * alpha[:, None], dst2, num_segments=_N,
                                indices_are_sorted=True)
        bprev = p["b"]
    return _call(
        _bias_relu_kernel,
        _H,
        (h, bprev.reshape(1, _H)),
        [_row_spec(_H), _full_spec(1, _H)],
    )


# speedup vs baseline: 1.2600x; 1.1995x over previous
import functools

import jax
import jax.numpy as jnp
from jax.experimental import pallas as pl

_N = 10000
_H = 512
_BN_EPS = 1e-5
_RT = 1000
_G = _N // _RT


def _row_spec(k):
    return pl.BlockSpec((_RT, k), lambda i: (i, 0))


def _full_spec(r, c):
    return pl.BlockSpec((r, c), lambda i: (0, 0))


def _gin_in_kernel(h_ref, a_ref, w_ref, b_ref, o_ref):
    y = h_ref[...] + a_ref[...]
    o_ref[...] = (
        jnp.dot(y, w_ref[...], preferred_element_type=jnp.float32) + b_ref[...]
    )


def _stats_kernel(z_ref, o_ref, *, relu):
    z = z_ref[...]
    if relu:
        z = jnp.maximum(z, 0.0)
    s = jnp.sum(z, axis=0, keepdims=True)
    ss = jnp.sum(z * z, axis=0, keepdims=True)
    val = jnp.concatenate([s, ss, jnp.zeros((6, z.shape[1]), jnp.float32)], axis=0)

    @pl.when(pl.program_id(0) == 0)
    def _():
        o_ref[...] = jnp.zeros_like(o_ref)

    o_ref[...] += val


def _bn_norm(z, st_ref, g, be):
    mu = st_ref[0:1, :] / _N
    var = st_ref[1:2, :] / _N - mu * mu
    inv = jax.lax.rsqrt(var + _BN_EPS)
    return g * (z - mu) * inv + be


def _gin_out_kernel(z_ref, st_ref, g_ref, be_ref, w2_ref, b2_ref, o_ref, *, relu_out):
    zn = _bn_norm(z_ref[...], st_ref, g_ref[...], be_ref[...])
    r = jnp.maximum(zn, 0.0)
    o = jnp.dot(r, w2_ref[...], preferred_element_type=jnp.float32) + b2_ref[...]
    if relu_out:
        o = jnp.maximum(o, 0.0)
    o_ref[...] = o


def _gat0_kernel(h_ref, st_ref, g_ref, be_ref, wc_ref, o_ref):
    r = jnp.maximum(h_ref[...], 0.0)
    z = _bn_norm(r, st_ref, g_ref[...], be_ref[...])
    o_ref[...] = jnp.dot(z, wc_ref[...], preferred_element_type=jnp.float32)


def _gat_kernel(h_ref, bp_ref, wc_ref, o_ref):
    z = jnp.maximum(h_ref[...] + bp_ref[...], 0.0)
    o_ref[...] = jnp.dot(z, wc_ref[...], preferred_element_type=jnp.float32)


def _bias_relu_kernel(h_ref, b_ref, o_ref):
    o_ref[...] = jnp.maximum(h_ref[...] + b_ref[...], 0.0)


def _call(kfn, out_cols, in_arrays, in_specs):
    return pl.pallas_call(
        kfn,
        grid=(_G,),
        in_specs=in_specs,
        out_specs=_row_spec(out_cols),
        out_shape=jax.ShapeDtypeStruct((_N, out_cols), jnp.float32),
    )(*in_arrays)


def _stats(z, relu):
    return pl.pallas_call(
        functools.partial(_stats_kernel, relu=relu),
        grid=(_G,),
        in_specs=[_row_spec(_H)],
        out_specs=_full_spec(8, _H),
        out_shape=jax.ShapeDtypeStruct((8, _H), jnp.float32),
    )(z)


def kernel(x, edge_index, edge_attr, edge_weight, batch, params):
    order = jnp.argsort(edge_index[1])
    src = edge_index[0][order]
    dst = edge_index[1][order]
    h = x
    for l in range(3):
        p = params["gin"][l]
        k = h.shape[1]
        agg = jax.ops.segment_sum(h[src], dst, num_segments=_N,
                                  indices_are_sorted=True)
        z = _call(
            _gin_in_kernel,
            _H,
            (h, agg, p["W1"], p["b1"].reshape(1, _H)),
            [_row_spec(k), _row_spec(k), _full_spec(k, _H), _full_spec(1, _H)],
        )
        st = _stats(z, relu=False)
        h = _call(
            functools.partial(_gin_out_kernel, relu_out=l < 2),
            _H,
            (z, st, p["g1"].reshape(1, _H), p["be1"].reshape(1, _H), p["W2"],
             p["b2"].reshape(1, _H)),
            [_row_spec(_H), _full_spec(8, _H), _full_spec(1, _H), _full_spec(1, _H),
             _full_spec(_H, _H), _full_spec(1, _H)],
        )

    st = _stats(h, relu=True)
    bprev = None
    for l in range(3):
        p = params["gat"][l]
        wc = jnp.concatenate(
            [p["W"], p["W"] @ p["as"][:, None], p["W"] @ p["ad"][:, None],
             jnp.zeros((_H, 126), jnp.float32)], axis=1)
        if l == 0:
            o = _call(
                _gat0_kernel,
                _H + 128,
                (h, st, params["bn1_g"].reshape(1, _H),
                 params["bn1_b"].reshape(1, _H), wc),
                [_row_spec(_H), _full_spec(8, _H), _full_spec(1, _H),
                 _full_spec(1, _H), _full_spec(_H, _H + 128)],
            )
        else:
            o = _call(
                _gat_kernel,
                _H + 128,
                (h, bprev.reshape(1, _H), wc),
                [_row_spec(_H), _full_spec(1, _H), _full_spec(_H, _H + 128)],
            )
        hw = o[:, :_H]
        a_src = o[:, _H]
        a_dst = o[:, _H + 1]
        e = jax.nn.leaky_relu(a_src[src] + a_dst[dst], 0.2)
        e_loop = jax.nn.leaky_relu(a_src + a_dst, 0.2)
        m = jax.ops.segment_max(e, dst, num_segments=_N, indices_are_sorted=True)
        m = jnp.maximum(m, e_loop)
        ex = jnp.exp(e - m[dst])
        ex_loop = jnp.exp(e_loop - m)
        s = jax.ops.segment_sum(ex, dst, num_segments=_N,
                                indices_are_sorted=True) + ex_loop
        alpha = ex / (s[dst] + 1e-16)
        alpha_loop = ex_loop / (s + 1e-16)
        h = jax.ops.segment_sum(hw[src] * alpha[:, None], dst, num_segments=_N,
                                indices_are_sorted=True) + hw * alpha_loop[:, None]
        bprev = p["b"]
    return _call(
        _bias_relu_kernel,
        _H,
        (h, bprev.reshape(1, _H)),
        [_row_spec(_H), _full_spec(1, _H)],
    )
